# baseline (speedup 1.0000x reference)
import functools

import jax
import jax.numpy as jnp
from jax.experimental import pallas as pl
from jax.experimental.pallas import tpu as pltpu
from jax.experimental.pallas import tpu_sc as plsc

_B = 1024
_C = 100000

_SC_LO = 0
_SC_HI = 98304
_SC_BLOCKS = (_SC_HI - _SC_LO) // 2048
_SC_CHUNKS = (_SC_HI - _SC_LO) // 512
_SC_OUT_W = _SC_BLOCKS * 128

_TC_ROWS = 32
_TAIL_BLOCK = _SC_HI // 2048
_TAIL_VALID = _C - _SC_HI

_WIN = 512
_CMB_ROWS = 128

_NEG = float("-inf")


def _sc_block_max(prediction, y32):
    mesh = plsc.VectorSubcoreMesh(core_axis_name="c", subcore_axis_name="s")

    @pl.kernel(
        out_type=jax.ShapeDtypeStruct((_B, _SC_OUT_W), jnp.float32),
        mesh=mesh,
    )
    def sc_kernel(p_hbm, y_hbm, b_hbm, o_hbm):
        def body(in_vmem, y_vmem, b_vmem, out_vmem):
            neg = jnp.full((16,), _NEG, jnp.float32)
            blk_base = b_vmem[0, pl.ds(0, 16)][0] * 2048
            lane16 = jax.lax.broadcasted_iota(jnp.int32, (16,), 0)

            @pl.loop(0, 8)
            def _(r):
                @pl.loop(0, 4)
                def _(j):
                    base = j * _WIN
                    acc = functools.reduce(
                        jnp.maximum,
                        [
                            in_vmem[r, pl.ds(base + 16 * s, 16)]
                            for s in range(_WIN // 16)
                        ],
                    )
                    out_vmem[r, pl.ds(j * 16, 16)] = acc

                out_vmem[r, pl.ds(64, 16)] = neg
                out_vmem[r, pl.ds(80, 16)] = neg
                out_vmem[r, pl.ds(96, 16)] = neg
                out_vmem[r, pl.ds(112, 16)] = neg

                y_loc = y_vmem[r, pl.ds(0, 16)][0] - blk_base

                @pl.when((y_loc >= 0) & (y_loc < 2048))
                def _():
                    a16 = (y_loc // 16) * 16
                    vy = in_vmem[r, pl.ds(a16, 16)]
                    out_vmem[r, pl.ds(64, 16)] = jnp.where(
                        lane16 == y_loc - a16, vy, neg
                    )
                    cbase = (y_loc // _WIN) * _WIN
                    l_in = y_loc - cbase
                    wm = functools.reduce(
                        jnp.maximum,
                        [
                            jnp.where(
                                lane16 == (l_in - 16 * s),
                                neg,
                                in_vmem[r, pl.ds(cbase + 16 * s, 16)],
                            )
                            for s in range(_WIN // 16)
                        ],
                    )
                    out_vmem[r, pl.ds(80, 16)] = wm

        pltpu.emit_pipeline(
            body,
            grid=(_B // 8, _SC_BLOCKS),
            in_specs=[
                pl.BlockSpec((8, 2048), index_map=lambda i, j: (i, j)),
                pl.BlockSpec((8, 16), index_map=lambda i, j: (i, 0)),
                pl.BlockSpec((1, 16), index_map=lambda i, j: (j, 0)),
            ],
            out_specs=[pl.BlockSpec((8, 128), index_map=lambda i, j: (i, j))],
            core_axis_name=("c", "s"),
            dimension_semantics=(pltpu.PARALLEL, pltpu.PARALLEL),
        )(p_hbm, y_hbm, b_hbm, o_hbm)

    y16 = jnp.broadcast_to(y32.reshape(_B, 1), (_B, 16))
    blk16 = jnp.broadcast_to(
        jnp.arange(_SC_BLOCKS, dtype=jnp.int32).reshape(_SC_BLOCKS, 1),
        (_SC_BLOCKS, 16),
    )
    return sc_kernel(prediction, y16, blk16)


def _tc_block(tail_ref, yv_ref, tcm_ref, cls_ref):
    yv = yv_ref[...]
    tail = tail_ref[...]
    col_t = jax.lax.broadcasted_iota(jnp.int32, (_TC_ROWS, 2048), 1)
    is_y = col_t == (yv - _SC_HI)
    bad_t = is_y | (col_t >= _TAIL_VALID)
    tcm_ref[...] = jnp.max(jnp.where(bad_t, _NEG, tail), axis=1, keepdims=True)
    cls_ref[...] = jnp.max(jnp.where(is_y, tail, _NEG), axis=1, keepdims=True)


def _combine_block(sc_ref, yv_ref, tcm_ref, clst_ref, out_ref):
    yv = yv_ref[...]
    sc = sc_ref[...]
    pos = jax.lax.broadcasted_iota(jnp.int32, (_CMB_ROWS, _SC_OUT_W), 1)
    t = pos % 128
    j_y = yv // _WIN
    chunk_of_pos = (pos // 128) * 4 + t // 16
    is_chunk = t < 64
    drop = is_chunk & (chunk_of_pos == j_y)
    sc_masked = jnp.max(
        jnp.where(drop | ~is_chunk, _NEG, sc), axis=1, keepdims=True
    )
    wm = jnp.max(jnp.where((t >= 80) & (t < 96), sc, _NEG), axis=1, keepdims=True)
    cls_sc = jnp.max(jnp.where((t >= 64) & (t < 80), sc, _NEG), axis=1, keepdims=True)
    target = jnp.maximum(jnp.maximum(sc_masked, tcm_ref[...]), wm)
    out_ref[...] = target - jnp.maximum(cls_sc, clst_ref[...])


def kernel(prediction, y):
    batch, num_classes = prediction.shape
    assert (batch, num_classes) == (_B, _C)
    y32 = y.astype(jnp.int32)
    y2 = y32.reshape(batch, 1)

    sc_out = _sc_block_max(prediction, y32)

    r = _TC_ROWS
    tcm, clst = pl.pallas_call(
        _tc_block,
        grid=(batch // r,),
        in_specs=[
            pl.BlockSpec((r, 2048), lambda i: (i, _TAIL_BLOCK)),
            pl.BlockSpec((r, 1), lambda i: (i, 0)),
        ],
        out_specs=[
            pl.BlockSpec((r, 1), lambda i: (i, 0)),
            pl.BlockSpec((r, 1), lambda i: (i, 0)),
        ],
        out_shape=[
            jax.ShapeDtypeStruct((batch, 1), jnp.float32),
            jax.ShapeDtypeStruct((batch, 1), jnp.float32),
        ],
    )(prediction, y2)

    rc = _CMB_ROWS
    out = pl.pallas_call(
        _combine_block,
        grid=(batch // rc,),
        in_specs=[
            pl.BlockSpec((rc, _SC_OUT_W), lambda i: (i, 0)),
            pl.BlockSpec((rc, 1), lambda i: (i, 0)),
            pl.BlockSpec((rc, 1), lambda i: (i, 0)),
            pl.BlockSpec((rc, 1), lambda i: (i, 0)),
        ],
        out_specs=pl.BlockSpec((rc, 1), lambda i: (i, 0)),
        out_shape=jax.ShapeDtypeStruct((batch, 1), jnp.float32),
    )(sc_out, y2, tcm, clst)
    return out.reshape(batch)


# speedup vs baseline: 1.2917x; 1.2917x over previous
import jax
import jax.numpy as jnp
from jax.experimental import pallas as pl
from jax.experimental.pallas import tpu as pltpu

_ROWS_PER_BLOCK = 32
_NUM_CHUNKS = 4
_WIN = 128


def _cw_loss_block(y_smem, *refs):
    chunk_refs = refs[:_NUM_CHUNKS]
    yv_ref = refs[_NUM_CHUNKS]
    out_ref = refs[_NUM_CHUNKS + 1]

    i = pl.program_id(0)
    r_rows = out_ref.shape[0]
    w = chunk_refs[0].shape[1]
    num_classes = _NUM_CLASSES
    max_q = w // _WIN - 1

    neg = jnp.float32(-jnp.inf)
    yv = yv_ref[...]
    lane = jax.lax.broadcasted_iota(jnp.int32, (r_rows, _WIN), 1)
    col = jax.lax.broadcasted_iota(jnp.int32, (r_rows, w), 1)

    m1 = None
    cls = None
    for k, p_ref in enumerate(chunk_refs):
        valid_w = min(w, num_classes - k * w)
        p = p_ref[...]
        if valid_w < w:
            p = jnp.where(col < valid_w, p, neg)
        mk = jnp.max(p, axis=1, keepdims=True)
        m1 = mk if m1 is None else jnp.maximum(m1, mk)

        wins = []
        for r in range(r_rows):
            yr = y_smem[i * r_rows + r] - k * w
            q = jnp.maximum(jnp.minimum(yr // _WIN, max_q), 0)
            wins.append(p_ref[pl.ds(r, 1), pl.ds(q * _WIN, _WIN)])
        win = jnp.concatenate(wins, axis=0)

        yl = yv - k * w
        q_v = jnp.clip(yl // _WIN, 0, max_q)
        off = yl - q_v * _WIN
        ck = jnp.max(jnp.where(lane == off, win, neg), axis=1, keepdims=True)
        cls = ck if cls is None else jnp.maximum(cls, ck)

    def fix():
        t = None
        for k, p_ref in enumerate(chunk_refs):
            valid_w = min(w, num_classes - k * w)
            bad = col == (yv - k * w)
            if valid_w < w:
                bad = bad | (col >= valid_w)
            tk = jnp.max(jnp.where(bad, neg, p_ref[...]), axis=1, keepdims=True)
            t = tk if t is None else jnp.maximum(t, tk)
        return t

    target = jax.lax.cond(jnp.any(cls >= m1), fix, lambda: m1)
    out_ref[...] = target - cls


_NUM_CLASSES = 100000


def kernel(prediction, y):
    batch, num_classes = prediction.shape
    assert num_classes == _NUM_CLASSES
    r = _ROWS_PER_BLOCK
    k = _NUM_CHUNKS
    w = -(-num_classes // k)
    w = -(-w // _WIN) * _WIN
    y32 = y.astype(jnp.int32)
    y2 = y32.reshape(batch, 1)
    chunk_specs = [
        pl.BlockSpec((r, w), lambda i, ys, kk=kk: (i, kk)) for kk in range(k)
    ]
    grid_spec = pltpu.PrefetchScalarGridSpec(
        num_scalar_prefetch=1,
        grid=(batch // r,),
        in_specs=chunk_specs + [pl.BlockSpec((r, 1), lambda i, ys: (i, 0))],
        out_specs=pl.BlockSpec((r, 1), lambda i, ys: (i, 0)),
    )
    out = pl.pallas_call(
        _cw_loss_block,
        grid_spec=grid_spec,
        out_shape=jax.ShapeDtypeStruct((batch, 1), jnp.float32),
    )(y32, *([prediction] * k), y2)
    return out.reshape(batch)
